# Initial kernel scaffold; baseline (speedup 1.0000x reference)
import jax
import jax.numpy as jnp
from jax.experimental import pallas as pl


def kernel(x, W_omega, b_omega, W_p1, b_p1, W_p2, b_p2, scale, W_mag, b_mag, W_q, b_q, ln_g, ln_b, W_o1, b_o1, W_o2, b_o2):
    raise NotImplementedError("write your pallas kernel here")


# speedup vs baseline: 2.1657x; 2.1657x over previous
import functools

import jax
import jax.numpy as jnp
from jax.experimental import pallas as pl
from jax.experimental.pallas import tpu as pltpu

CHUNK = 256


def _erf(z):
    a1, a2, a3, a4, a5 = (0.254829592, -0.284496736, 1.421413741,
                          -1.453152027, 1.061405429)
    az = jnp.abs(z)
    t = 1.0 / (1.0 + 0.3275911 * az)
    poly = ((((a5 * t + a4) * t + a3) * t + a2) * t + a1) * t
    e = 1.0 - poly * jnp.exp(-az * az)
    return jnp.where(z < 0, -e, e)


def _gelu_exact(t):
    return 0.5 * t * (1.0 + _erf(t * 0.7071067811865476))


def _psi_kernel(tri_ref, x_ref, W_omega, b_omega, W_p1, b_p1, W_p2, b_p2,
                scale, W_mag, b_mag, W_q, b_q, ln_g, ln_b, W_o1, b_o1,
                W_o2, b_o2, o_ref, c_om, c_r, c_i, c_m):
    c = pl.program_id(1)

    @pl.when(c == 0)
    def _():
        c_om[...] = jnp.zeros_like(c_om)
        c_r[...] = jnp.zeros_like(c_r)
        c_i[...] = jnp.zeros_like(c_i)
        c_m[...] = jnp.zeros_like(c_m)

    xb = x_ref[0]
    tri = tri_ref[...]

    def dot(a, b):
        return jnp.dot(a, b, preferred_element_type=jnp.float32)

    omega = dot(xb, W_omega[...]) + b_omega[...]
    mag = jax.nn.sigmoid(dot(xb, W_mag[...]) + b_mag[...]) * 5.0
    h1 = _gelu_exact(dot(xb, W_p1[...]) + b_p1[...])
    phi_init = dot(h1, W_p2[...]) + b_p2[...]

    so = omega * jnp.abs(scale[...])
    phi = phi_init + dot(tri, so) + c_om[...]
    c_om[...] += jnp.sum(so, axis=0, keepdims=True)

    cos_phi = jnp.cos(phi)
    sin_phi = jnp.sin(phi)

    wc = mag * xb
    tr = wc * cos_phi
    ti = wc * sin_phi
    mem_r = dot(tri, tr) + c_r[...]
    c_r[...] += jnp.sum(tr, axis=0, keepdims=True)
    mem_i = dot(tri, ti) + c_i[...]
    c_i[...] += jnp.sum(ti, axis=0, keepdims=True)
    cmag = dot(tri, mag) + c_m[...]
    c_m[...] += jnp.sum(mag, axis=0, keepdims=True)

    inv_sq = jax.lax.rsqrt(cmag + 1e-8)
    mr = mem_r * inv_sq
    mi = mem_i * inv_sq

    phi_q = phi + dot(xb, W_q[...]) + b_q[...]
    cq = jnp.cos(phi_q)
    sq = jnp.sin(phi_q)
    ret_r = mr * cq + mi * sq
    ret_i = mi * cq - mr * sq

    ctx = jnp.concatenate([xb * cos_phi, xb * sin_phi, ret_r, ret_i], axis=-1)
    mu = jnp.mean(ctx, axis=-1, keepdims=True)
    xc = ctx - mu
    var = jnp.mean(xc * xc, axis=-1, keepdims=True)
    ln = xc * jax.lax.rsqrt(var + 1e-5) * ln_g[...] + ln_b[...]

    h = _gelu_exact(dot(ln, W_o1[...]) + b_o1[...])
    o_ref[0] = xb + dot(h, W_o2[...]) + b_o2[...]


@jax.jit
def kernel(x, W_omega, b_omega, W_p1, b_p1, W_p2, b_p2, scale, W_mag, b_mag,
           W_q, b_q, ln_g, ln_b, W_o1, b_o1, W_o2, b_o2):
    B, S, D = x.shape
    C = CHUNK if S % CHUNK == 0 else S
    NC = S // C

    tri = jnp.tril(jnp.ones((C, C), jnp.float32))

    row = lambda v: v.reshape(1, -1)
    const = lambda shape: pl.BlockSpec(shape, lambda b, c: (0,) * len(shape))

    grid = (B, NC)
    out = pl.pallas_call(
        _psi_kernel,
        out_shape=jax.ShapeDtypeStruct((B, S, D), jnp.float32),
        grid=grid,
        in_specs=[
            const((C, C)),
            pl.BlockSpec((1, C, D), lambda b, c: (b, c, 0)),
            const((D, D)), const((1, D)),
            const((D, D)), const((1, D)),
            const((D, D)), const((1, D)),
            const((1, D)),
            const((D, D)), const((1, D)),
            const((D, D)), const((1, D)),
            const((1, 4 * D)), const((1, 4 * D)),
            const((4 * D, 2 * D)), const((1, 2 * D)),
            const((2 * D, D)), const((1, D)),
        ],
        out_specs=pl.BlockSpec((1, C, D), lambda b, c: (b, c, 0)),
        scratch_shapes=[pltpu.VMEM((1, D), jnp.float32)] * 4,
        compiler_params=pltpu.CompilerParams(
            dimension_semantics=("parallel", "arbitrary"),
            vmem_limit_bytes=100 * 1024 * 1024,
        ),
        name="psi_no_gate",
    )(tri, x, W_omega, row(b_omega), W_p1, row(b_p1), W_p2, row(b_p2),
      row(scale), W_mag, row(b_mag), W_q, row(b_q), row(ln_g), row(ln_b),
      W_o1, row(b_o1), W_o2, row(b_o2))
    return out


# speedup vs baseline: 3.0572x; 1.4117x over previous
import functools

import jax
import jax.numpy as jnp
from jax.experimental import pallas as pl
from jax.experimental.pallas import tpu as pltpu

CHUNK = 256


def _erf(z):
    a1, a2, a3, a4, a5 = (0.254829592, -0.284496736, 1.421413741,
                          -1.453152027, 1.061405429)
    az = jnp.abs(z)
    t = 1.0 / (1.0 + 0.3275911 * az)
    poly = ((((a5 * t + a4) * t + a3) * t + a2) * t + a1) * t
    e = 1.0 - poly * jnp.exp(-az * az)
    return jnp.where(z < 0, -e, e)


def _gelu_exact(t):
    return 0.5 * t * (1.0 + _erf(t * 0.7071067811865476))


_PIO2_HI = 1.57079637e0
_PIO2_LO = -4.37113883e-8


def _sincos(x):
    k_i = jnp.round(x * (2.0 / jnp.pi)).astype(jnp.int32)
    kf = k_i.astype(jnp.float32)
    r = (x - kf * _PIO2_HI) - kf * _PIO2_LO
    r2 = r * r
    sin_r = ((-1.9515295891e-4 * r2 + 8.3321608736e-3) * r2
             - 1.6666654611e-1) * r2 * r + r
    cos_r = ((-1.388731625493765e-3 * r2 + 4.16666456e-2) * r2
             - 0.5) * r2 + 1.0
    swap = (k_i & 1) != 0
    sin_base = jnp.where(swap, cos_r, sin_r)
    cos_base = jnp.where(swap, sin_r, cos_r)
    sin_neg = (k_i & 2) != 0
    cos_neg = ((k_i + 1) & 2) != 0
    sin_x = jnp.where(sin_neg, -sin_base, sin_base)
    cos_x = jnp.where(cos_neg, -cos_base, cos_base)
    return sin_x, cos_x


def _psi_kernel(tri_ref, x_ref, W_omega, b_omega, W_p1, b_p1, W_p2, b_p2,
                scale, W_mag, b_mag, W_q, b_q, ln_g, ln_b, W_o1, b_o1,
                W_o2, b_o2, o_ref, c_om, c_r, c_i, c_m):
    c = pl.program_id(1)

    @pl.when(c == 0)
    def _():
        c_om[...] = jnp.zeros_like(c_om)
        c_r[...] = jnp.zeros_like(c_r)
        c_i[...] = jnp.zeros_like(c_i)
        c_m[...] = jnp.zeros_like(c_m)

    xb = x_ref[0]
    tri = tri_ref[...]

    def dot(a, b):
        return jnp.dot(a, b, preferred_element_type=jnp.float32)

    omega = dot(xb, W_omega[...]) + b_omega[...]
    mag = jax.nn.sigmoid(dot(xb, W_mag[...]) + b_mag[...]) * 5.0
    h1 = _gelu_exact(dot(xb, W_p1[...]) + b_p1[...])
    phi_init = dot(h1, W_p2[...]) + b_p2[...]

    so = omega * jnp.abs(scale[...])
    phi = phi_init + dot(tri, so) + c_om[...]
    c_om[...] += jnp.sum(so, axis=0, keepdims=True)

    sin_phi, cos_phi = _sincos(phi)

    wc = mag * xb
    tr = wc * cos_phi
    ti = wc * sin_phi
    mem_r = dot(tri, tr) + c_r[...]
    c_r[...] += jnp.sum(tr, axis=0, keepdims=True)
    mem_i = dot(tri, ti) + c_i[...]
    c_i[...] += jnp.sum(ti, axis=0, keepdims=True)
    cmag = dot(tri, mag) + c_m[...]
    c_m[...] += jnp.sum(mag, axis=0, keepdims=True)

    inv_sq = jax.lax.rsqrt(cmag + 1e-8)
    mr = mem_r * inv_sq
    mi = mem_i * inv_sq

    phi_q = phi + dot(xb, W_q[...]) + b_q[...]
    sq, cq = _sincos(phi_q)
    ret_r = mr * cq + mi * sq
    ret_i = mi * cq - mr * sq

    ctx = jnp.concatenate([xb * cos_phi, xb * sin_phi, ret_r, ret_i], axis=-1)
    mu = jnp.mean(ctx, axis=-1, keepdims=True)
    xc = ctx - mu
    var = jnp.mean(xc * xc, axis=-1, keepdims=True)
    ln = xc * jax.lax.rsqrt(var + 1e-5) * ln_g[...] + ln_b[...]

    h = _gelu_exact(dot(ln, W_o1[...]) + b_o1[...])
    o_ref[0] = xb + dot(h, W_o2[...]) + b_o2[...]


@jax.jit
def kernel(x, W_omega, b_omega, W_p1, b_p1, W_p2, b_p2, scale, W_mag, b_mag,
           W_q, b_q, ln_g, ln_b, W_o1, b_o1, W_o2, b_o2):
    B, S, D = x.shape
    C = CHUNK if S % CHUNK == 0 else S
    NC = S // C

    tri = jnp.tril(jnp.ones((C, C), jnp.float32))

    row = lambda v: v.reshape(1, -1)
    const = lambda shape: pl.BlockSpec(shape, lambda b, c: (0,) * len(shape))

    grid = (B, NC)
    out = pl.pallas_call(
        _psi_kernel,
        out_shape=jax.ShapeDtypeStruct((B, S, D), jnp.float32),
        grid=grid,
        in_specs=[
            const((C, C)),
            pl.BlockSpec((1, C, D), lambda b, c: (b, c, 0)),
            const((D, D)), const((1, D)),
            const((D, D)), const((1, D)),
            const((D, D)), const((1, D)),
            const((1, D)),
            const((D, D)), const((1, D)),
            const((D, D)), const((1, D)),
            const((1, 4 * D)), const((1, 4 * D)),
            const((4 * D, 2 * D)), const((1, 2 * D)),
            const((2 * D, D)), const((1, D)),
        ],
        out_specs=pl.BlockSpec((1, C, D), lambda b, c: (b, c, 0)),
        scratch_shapes=[pltpu.VMEM((1, D), jnp.float32)] * 4,
        compiler_params=pltpu.CompilerParams(
            dimension_semantics=("parallel", "arbitrary"),
            vmem_limit_bytes=100 * 1024 * 1024,
        ),
        name="psi_no_gate",
    )(tri, x, W_omega, row(b_omega), W_p1, row(b_p1), W_p2, row(b_p2),
      row(scale), W_mag, row(b_mag), W_q, row(b_q), row(ln_g), row(ln_b),
      W_o1, row(b_o1), W_o2, row(b_o2))
    return out
